# Initial kernel scaffold; baseline (speedup 1.0000x reference)
import jax
import jax.numpy as jnp
from jax.experimental import pallas as pl


def kernel(x, edge_index, ln_g1, ln_b1, Ws1, Wb1, bb1, bias1, ln_g2, ln_b2, Ws2, Wb2, bb2, bias2, ln_g3, ln_b3, Ws3, Wb3, bb3, bias3, bn_g, bn_b):
    raise NotImplementedError("write your pallas kernel here")


# speedup vs baseline: 19.9496x; 19.9496x over previous
import functools

import jax
import jax.numpy as jnp
from jax import lax
from jax.experimental import pallas as pl
from jax.experimental.pallas import tpu as pltpu
from jax.experimental.pallas import tpu_sc as plsc

_N = 10000
_E = 320000
_D_IN = 128
_HID = 128
_NCLS = 16
_GRIDS = 4
_H = 4.0 / (_GRIDS - 1)
_GRIDPTS = tuple(-2.0 + i * _H for i in range(_GRIDS))
_EPS = 1e-5

_NC = 2
_NS = 16
_EB = 80
_NW = _NC * _NS
_TB_W = _E // (_NW * _EB)
_TB_S = _E // (_NS * _EB)
_NBUF = 5
_NPAD = 10240
_RPT = _NPAD // _NS
_ZR = 128

_BR = 1000
_GR = _N // _BR


def _fill2d(ref, rows, cols, value):
    v = jnp.full((16,), value, jnp.float32)
    per_row = cols // 16

    def body(i, carry):
        r = i // per_row
        col = (i % per_row) * 16
        ref[r, pl.ds(col, 16)] = v
        return carry

    lax.fori_loop(0, rows * per_row, body, 0)


def _make_degree_kernel():
    mesh = plsc.VectorSubcoreMesh(core_axis_name="c", subcore_axis_name="s")

    @functools.partial(
        pl.kernel,
        out_type=jax.ShapeDtypeStruct((2 * _NPAD, 16), jnp.float32),
        mesh=mesh,
        compiler_params=pltpu.CompilerParams(use_tc_tiling_on_sc=False),
        scratch_types=[
            pltpu.VMEM((_TB_W, _EB), jnp.int32),
            pltpu.VMEM((_EB, 16), jnp.float32),
            pltpu.VMEM((_ZR, 16), jnp.float32),
            pltpu.VMEM_SHARED((_NPAD, 16), jnp.float32),
            pltpu.SemaphoreType.DMA,
        ],
    )
    def deg_kernel(col_hbm, out_hbm, colv, ones_v, zv, acc, ssem):
        c = lax.axis_index("c")
        s = lax.axis_index("s")
        w = c * _NS + s
        _fill2d(ones_v, _EB, 16, 1.0)
        _fill2d(zv, _ZR, 16, 0.0)
        for j in range(5):
            pltpu.sync_copy(zv, acc.at[pl.ds(s * _RPT + j * _ZR, _ZR)])
        pltpu.sync_copy(col_hbm.at[w], colv)
        plsc.subcore_barrier()

        def wave(i, carry):
            g = i * _NBUF
            hs = [
                pltpu.async_copy(ones_v, acc.at[colv.at[g + j]], ssem, add=True)
                for j in range(_NBUF)
            ]
            for h in hs:
                h.wait()
            return carry

        lax.fori_loop(0, _TB_W // _NBUF, wave, 0)
        plsc.subcore_barrier()
        for j in range(5):
            r0 = s * _RPT + j * _ZR
            pltpu.sync_copy(acc.at[pl.ds(r0, _ZR)],
                            out_hbm.at[pl.ds(c * _NPAD + r0, _ZR)])

    return deg_kernel


def _make_edge_scatter_split():
    mesh = plsc.VectorSubcoreMesh(core_axis_name="c", subcore_axis_name="s")

    @functools.partial(
        pl.kernel,
        out_type=jax.ShapeDtypeStruct((2 * _NPAD, 64), jnp.float32),
        mesh=mesh,
        compiler_params=pltpu.CompilerParams(use_tc_tiling_on_sc=False),
        scratch_types=[
            pltpu.VMEM((_TB_S, _EB), jnp.int32),
            pltpu.VMEM((_TB_S, _EB), jnp.int32),
            [pltpu.VMEM((_EB, 64), jnp.float32) for _ in range(_NBUF)],
            pltpu.VMEM((_ZR, 64), jnp.float32),
            pltpu.VMEM_SHARED((_NPAD, 64), jnp.float32),
            pltpu.SemaphoreType.DMA,
            pltpu.SemaphoreType.DMA,
        ],
    )
    def edge_kernel(ul_hbm, ur_hbm, row_hbm, col_hbm, out_hbm,
                    rowv, colv, bufs, zv, acc, gsem, ssem):
        c = lax.axis_index("c")
        s = lax.axis_index("s")
        _fill2d(zv, _ZR, 64, 0.0)
        for j in range(5):
            pltpu.sync_copy(zv, acc.at[pl.ds(s * _RPT + j * _ZR, _ZR)])
        pltpu.sync_copy(row_hbm.at[s], rowv)
        pltpu.sync_copy(col_hbm.at[s], colv)
        plsc.subcore_barrier()

        def run(u_half):
            def wave(i, carry):
                g = i * _NBUF
                gh = [
                    pltpu.async_copy(u_half.at[rowv.at[g + j]], bufs[j], gsem)
                    for j in range(_NBUF)
                ]
                for h in gh:
                    h.wait()
                sh = [
                    pltpu.async_copy(bufs[j], acc.at[colv.at[g + j]], ssem,
                                     add=True)
                    for j in range(_NBUF)
                ]
                for h in sh:
                    h.wait()
                return carry

            lax.fori_loop(0, _TB_S // _NBUF, wave, 0)

        @pl.when(c == 0)
        def _():
            run(ul_hbm)

        @pl.when(c == 1)
        def _():
            run(ur_hbm)

        plsc.subcore_barrier()
        for j in range(5):
            r0 = s * _RPT + j * _ZR
            pltpu.sync_copy(acc.at[pl.ds(r0, _ZR)],
                            out_hbm.at[pl.ds(c * _NPAD + r0, _ZR)])

    return edge_kernel


def _make_edge_scatter16():
    mesh = plsc.VectorSubcoreMesh(core_axis_name="c", subcore_axis_name="s")

    @functools.partial(
        pl.kernel,
        out_type=jax.ShapeDtypeStruct((2 * _NPAD, 16), jnp.float32),
        mesh=mesh,
        compiler_params=pltpu.CompilerParams(use_tc_tiling_on_sc=False),
        scratch_types=[
            pltpu.VMEM((_TB_W, _EB), jnp.int32),
            pltpu.VMEM((_TB_W, _EB), jnp.int32),
            [pltpu.VMEM((_EB, 16), jnp.float32) for _ in range(_NBUF)],
            pltpu.VMEM((_ZR, 16), jnp.float32),
            pltpu.VMEM_SHARED((_NPAD, 16), jnp.float32),
            pltpu.SemaphoreType.DMA,
            pltpu.SemaphoreType.DMA,
        ],
    )
    def edge_kernel(u_hbm, row_hbm, col_hbm, out_hbm,
                    rowv, colv, bufs, zv, acc, gsem, ssem):
        c = lax.axis_index("c")
        s = lax.axis_index("s")
        w = c * _NS + s
        _fill2d(zv, _ZR, 16, 0.0)
        for j in range(5):
            pltpu.sync_copy(zv, acc.at[pl.ds(s * _RPT + j * _ZR, _ZR)])
        pltpu.sync_copy(row_hbm.at[w], rowv)
        pltpu.sync_copy(col_hbm.at[w], colv)
        plsc.subcore_barrier()

        def wave(i, carry):
            g = i * _NBUF
            gh = [
                pltpu.async_copy(u_hbm.at[rowv.at[g + j]], bufs[j], gsem)
                for j in range(_NBUF)
            ]
            for h in gh:
                h.wait()
            sh = [
                pltpu.async_copy(bufs[j], acc.at[colv.at[g + j]], ssem, add=True)
                for j in range(_NBUF)
            ]
            for h in sh:
                h.wait()
            return carry

        lax.fori_loop(0, _TB_W // _NBUF, wave, 0)
        plsc.subcore_barrier()
        for j in range(5):
            r0 = s * _RPT + j * _ZR
            pltpu.sync_copy(acc.at[pl.ds(r0, _ZR)],
                            out_hbm.at[pl.ds(c * _NPAD + r0, _ZR)])

    return edge_kernel


_make_degree_kernel = functools.cache(_make_degree_kernel)
_make_edge_scatter_split = functools.cache(_make_edge_scatter_split)
_make_edge_scatter16 = functools.cache(_make_edge_scatter16)


def _dis_from_deg(d0_blk, d1_blk):
    deg = d0_blk[:, 0:1] + d1_blk[:, 0:1] + 1.0
    return lax.rsqrt(deg)


def _kan_math(h, lng, lnb, wg, wbt, bb):
    m = jnp.mean(h, axis=-1, keepdims=True)
    v = jnp.mean((h - m) ** 2, axis=-1, keepdims=True)
    y = (h - m) * lax.rsqrt(v + _EPS) * lng + lnb
    acc = jnp.dot(h * jax.nn.sigmoid(h), wbt,
                  preferred_element_type=jnp.float32) + bb
    for g in range(_GRIDS):
        bg = jnp.exp(-(((y - _GRIDPTS[g]) * (1.0 / _H)) ** 2))
        acc = acc + jnp.dot(bg, wg[g], preferred_element_type=jnp.float32)
    return acc


def _full_spec(shape):
    n = len(shape)
    return pl.BlockSpec(shape, lambda i, _n=n: (0,) * _n)


def _row_spec(width):
    return pl.BlockSpec((_BR, width), lambda i: (i, 0))


def _tc_kan_first(x, d0, d1, lng, lnb, wg, wbt, bb):
    def body(x_ref, d0_ref, d1_ref, lng_ref, lnb_ref, wg_ref, wbt_ref, bb_ref,
             ul_ref, ur_ref):
        dis = _dis_from_deg(d0_ref[...], d1_ref[...])
        t = _kan_math(x_ref[...], lng_ref[...], lnb_ref[...], wg_ref[...],
                      wbt_ref[...], bb_ref[...])
        u = t * dis
        ul_ref[...] = u[:, :64]
        ur_ref[...] = u[:, 64:]

    return pl.pallas_call(
        body,
        grid=(_GR,),
        in_specs=[
            _row_spec(_D_IN), _row_spec(16), _row_spec(16),
            _full_spec(lng.shape), _full_spec(lnb.shape),
            _full_spec(wg.shape), _full_spec(wbt.shape), _full_spec(bb.shape),
        ],
        out_specs=[_row_spec(64), _row_spec(64)],
        out_shape=[jax.ShapeDtypeStruct((_N, 64), jnp.float32),
                   jax.ShapeDtypeStruct((_N, 64), jnp.float32)],
    )(x, d0, d1, lng, lnb, wg, wbt, bb)


def _tc_combine_stats(ul, ur, sp, d0, d1, bias):
    def body(ul_ref, ur_ref, sl_ref, sr_ref, d0_ref, d1_ref, b_ref,
             h_ref, st_ref):
        i = pl.program_id(0)
        dis = _dis_from_deg(d0_ref[...], d1_ref[...])
        su = jnp.concatenate(
            [sl_ref[...] + ul_ref[...], sr_ref[...] + ur_ref[...]], axis=1)
        h = su * dis + b_ref[...]
        h_ref[...] = h
        new = jnp.concatenate(
            [jnp.sum(h, axis=0, keepdims=True),
             jnp.sum(h * h, axis=0, keepdims=True)], axis=0)

        @pl.when(i == 0)
        def _():
            st_ref[...] = new

        @pl.when(i != 0)
        def _():
            st_ref[...] = st_ref[...] + new

    return pl.pallas_call(
        body,
        grid=(_GR,),
        in_specs=[
            _row_spec(64), _row_spec(64), _row_spec(64), _row_spec(64),
            _row_spec(16), _row_spec(16), _full_spec(bias.shape),
        ],
        out_specs=[_row_spec(_HID), _full_spec((2, _HID))],
        out_shape=[jax.ShapeDtypeStruct((_N, _HID), jnp.float32),
                   jax.ShapeDtypeStruct((2, _HID), jnp.float32)],
    )(ul, ur, sp[:_N], sp[_NPAD:_NPAD + _N], d0, d1, bias)


def _tc_bn_kan(h_pre, st, d0, d1, lng, lnb, wg, wbt, bb, bng, bnb):
    def body(hp_ref, st_ref, d0_ref, d1_ref, lng_ref, lnb_ref, wg_ref,
             wbt_ref, bb_ref, bng_ref, bnb_ref, ht_ref, ul_ref, ur_ref):
        st_v = st_ref[...]
        m = st_v[0:1, :] * (1.0 / _N)
        var = st_v[1:2, :] * (1.0 / _N) - m * m
        ht = (hp_ref[...] - m) * (bng_ref[...] * lax.rsqrt(var + _EPS)) + bnb_ref[...]
        ht_ref[...] = ht
        dis = _dis_from_deg(d0_ref[...], d1_ref[...])
        t = _kan_math(ht, lng_ref[...], lnb_ref[...], wg_ref[...], wbt_ref[...],
                      bb_ref[...])
        u = t * dis
        ul_ref[...] = u[:, :64]
        ur_ref[...] = u[:, 64:]

    return pl.pallas_call(
        body,
        grid=(_GR,),
        in_specs=[
            _row_spec(_HID), _full_spec((2, _HID)), _row_spec(16), _row_spec(16),
            _full_spec(lng.shape), _full_spec(lnb.shape),
            _full_spec(wg.shape), _full_spec(wbt.shape), _full_spec(bb.shape),
            _full_spec(bng.shape), _full_spec(bnb.shape),
        ],
        out_specs=[_row_spec(_HID), _row_spec(64), _row_spec(64)],
        out_shape=[jax.ShapeDtypeStruct((_N, _HID), jnp.float32),
                   jax.ShapeDtypeStruct((_N, 64), jnp.float32),
                   jax.ShapeDtypeStruct((_N, 64), jnp.float32)],
    )(h_pre, st, d0, d1, lng, lnb, wg, wbt, bb, bng, bnb)


def _tc_kan_concat(x, h1, h2_pre, st2, d0, d1, lng, lnb, wg, wbt, bb, bng, bnb):
    def body(x_ref, h1_ref, hp_ref, st_ref, d0_ref, d1_ref, lng_ref, lnb_ref,
             wg_ref, wbt_ref, bb_ref, bng_ref, bnb_ref, u_ref):
        st_v = st_ref[...]
        m = st_v[0:1, :] * (1.0 / _N)
        var = st_v[1:2, :] * (1.0 / _N) - m * m
        ht2 = (hp_ref[...] - m) * (bng_ref[...] * lax.rsqrt(var + _EPS)) + bnb_ref[...]
        h = jnp.concatenate([x_ref[...], h1_ref[...], ht2], axis=1)
        dis = _dis_from_deg(d0_ref[...], d1_ref[...])
        t = _kan_math(h, lng_ref[...], lnb_ref[...], wg_ref[...], wbt_ref[...],
                      bb_ref[...])
        u_ref[...] = t * dis

    return pl.pallas_call(
        body,
        grid=(_GR,),
        in_specs=[
            _row_spec(_D_IN), _row_spec(_HID), _row_spec(_HID),
            _full_spec((2, _HID)), _row_spec(16), _row_spec(16),
            _full_spec(lng.shape), _full_spec(lnb.shape),
            _full_spec(wg.shape), _full_spec(wbt.shape), _full_spec(bb.shape),
            _full_spec(bng.shape), _full_spec(bnb.shape),
        ],
        out_specs=_row_spec(_NCLS),
        out_shape=jax.ShapeDtypeStruct((_N, _NCLS), jnp.float32),
    )(x, h1, h2_pre, st2, d0, d1, lng, lnb, wg, wbt, bb, bng, bnb)


def _tc_combine_final(u, sp, d0, d1, bias):
    def body(u_ref, s0_ref, s1_ref, d0_ref, d1_ref, b_ref, o_ref):
        dis = _dis_from_deg(d0_ref[...], d1_ref[...])
        o_ref[...] = (s0_ref[...] + s1_ref[...] + u_ref[...]) * dis + b_ref[...]

    return pl.pallas_call(
        body,
        grid=(_GR,),
        in_specs=[
            _row_spec(_NCLS), _row_spec(_NCLS), _row_spec(_NCLS),
            _row_spec(16), _row_spec(16), _full_spec(bias.shape),
        ],
        out_specs=_row_spec(_NCLS),
        out_shape=jax.ShapeDtypeStruct((_N, _NCLS), jnp.float32),
    )(u, sp[:_N], sp[_NPAD:_NPAD + _N], d0, d1, bias)


def _prep_kan_weights(Ws, Wb, din, dout):
    wg = Ws.reshape(dout, din, _GRIDS).transpose(2, 1, 0)
    return wg, Wb.T


def kernel(x, edge_index, ln_g1, ln_b1, Ws1, Wb1, bb1, bias1,
           ln_g2, ln_b2, Ws2, Wb2, bb2, bias2,
           ln_g3, ln_b3, Ws3, Wb3, bb3, bias3, bn_g, bn_b):
    row_w = edge_index[0].reshape(_NW, _TB_W, _EB)
    col_w = edge_index[1].reshape(_NW, _TB_W, _EB)
    row_s = edge_index[0].reshape(_NS, _TB_S, _EB)
    col_s = edge_index[1].reshape(_NS, _TB_S, _EB)

    wg1, wbt1 = _prep_kan_weights(Ws1, Wb1, _D_IN, _HID)
    wg2, wbt2 = _prep_kan_weights(Ws2, Wb2, _HID, _HID)
    wg3, wbt3 = _prep_kan_weights(Ws3, Wb3, _D_IN + 2 * _HID, _NCLS)
    lng1, lnb1 = ln_g1.reshape(1, -1), ln_b1.reshape(1, -1)
    lng2, lnb2 = ln_g2.reshape(1, -1), ln_b2.reshape(1, -1)
    lng3, lnb3 = ln_g3.reshape(1, -1), ln_b3.reshape(1, -1)
    bbr1, bbr2, bbr3 = bb1.reshape(1, -1), bb2.reshape(1, -1), bb3.reshape(1, -1)
    br1, br2, br3 = bias1.reshape(1, -1), bias2.reshape(1, -1), bias3.reshape(1, -1)
    bng, bnb = bn_g.reshape(1, -1), bn_b.reshape(1, -1)

    degp = _make_degree_kernel()(col_w)
    d0, d1 = degp[:_N], degp[_NPAD:_NPAD + _N]

    ul1, ur1 = _tc_kan_first(x, d0, d1, lng1, lnb1, wg1, wbt1, bbr1)
    sp1 = _make_edge_scatter_split()(ul1, ur1, row_s, col_s)
    h1_pre, st1 = _tc_combine_stats(ul1, ur1, sp1, d0, d1, br1)

    h1, ul2, ur2 = _tc_bn_kan(h1_pre, st1, d0, d1, lng2, lnb2, wg2, wbt2,
                              bbr2, bng, bnb)
    sp2 = _make_edge_scatter_split()(ul2, ur2, row_s, col_s)
    h2_pre, st2 = _tc_combine_stats(ul2, ur2, sp2, d0, d1, br2)

    u3 = _tc_kan_concat(x, h1, h2_pre, st2, d0, d1, lng3, lnb3, wg3, wbt3,
                        bbr3, bng, bnb)
    sp3 = _make_edge_scatter16()(u3, row_w, col_w)
    return _tc_combine_final(u3, sp3, d0, d1, br3)


# speedup vs baseline: 20.5061x; 1.0279x over previous
import functools

import jax
import jax.numpy as jnp
from jax import lax
from jax.experimental import pallas as pl
from jax.experimental.pallas import tpu as pltpu
from jax.experimental.pallas import tpu_sc as plsc

_N = 10000
_E = 320000
_D_IN = 128
_HID = 128
_NCLS = 16
_GRIDS = 4
_H = 4.0 / (_GRIDS - 1)
_GRIDPTS = tuple(-2.0 + i * _H for i in range(_GRIDS))
_EPS = 1e-5

_NC = 2
_NS = 16
_EBS = 80
_EBW = 80
_NW = _NC * _NS
_TB_W = _E // (_NW * _EBW)
_TB_S = _E // (_NS * _EBS)
_NBUF = 5
_NPAD = 10240
_RPT = _NPAD // _NS
_ZR = 128

_BR = 1000
_GR = _N // _BR


def _fill2d(ref, rows, cols, value):
    v = jnp.full((16,), value, jnp.float32)
    per_row = cols // 16

    def body(i, carry):
        r = i // per_row
        col = (i % per_row) * 16
        ref[r, pl.ds(col, 16)] = v
        return carry

    lax.fori_loop(0, rows * per_row, body, 0)


def _make_degree_kernel():
    mesh = plsc.VectorSubcoreMesh(core_axis_name="c", subcore_axis_name="s")

    @functools.partial(
        pl.kernel,
        out_type=jax.ShapeDtypeStruct((2 * _NPAD, 16), jnp.float32),
        mesh=mesh,
        compiler_params=pltpu.CompilerParams(use_tc_tiling_on_sc=False),
        scratch_types=[
            pltpu.VMEM((_TB_W, _EBW), jnp.int32),
            pltpu.VMEM((_EBW, 16), jnp.float32),
            pltpu.VMEM((_ZR, 16), jnp.float32),
            pltpu.VMEM_SHARED((_NPAD, 16), jnp.float32),
            pltpu.SemaphoreType.DMA,
        ],
    )
    def deg_kernel(col_hbm, out_hbm, colv, ones_v, zv, acc, ssem):
        c = lax.axis_index("c")
        s = lax.axis_index("s")
        w = c * _NS + s
        _fill2d(ones_v, _EBW, 16, 1.0)
        _fill2d(zv, _ZR, 16, 0.0)
        for j in range(5):
            pltpu.sync_copy(zv, acc.at[pl.ds(s * _RPT + j * _ZR, _ZR)])
        pltpu.sync_copy(col_hbm.at[w], colv)
        plsc.subcore_barrier()

        def wave(i, carry):
            g = i * _NBUF
            hs = [
                pltpu.async_copy(ones_v, acc.at[colv.at[g + j]], ssem, add=True)
                for j in range(_NBUF)
            ]
            for h in hs:
                h.wait()
            return carry

        lax.fori_loop(0, _TB_W // _NBUF, wave, 0)
        plsc.subcore_barrier()
        for j in range(5):
            r0 = s * _RPT + j * _ZR
            pltpu.sync_copy(acc.at[pl.ds(r0, _ZR)],
                            out_hbm.at[pl.ds(c * _NPAD + r0, _ZR)])

    return deg_kernel


def _make_edge_scatter_split():
    mesh = plsc.VectorSubcoreMesh(core_axis_name="c", subcore_axis_name="s")

    @functools.partial(
        pl.kernel,
        out_type=jax.ShapeDtypeStruct((2 * _NPAD, 64), jnp.float32),
        mesh=mesh,
        compiler_params=pltpu.CompilerParams(use_tc_tiling_on_sc=False),
        scratch_types=[
            pltpu.VMEM((_TB_S, _EBS), jnp.int32),
            pltpu.VMEM((_TB_S, _EBS), jnp.int32),
            [pltpu.VMEM((_EBS, 64), jnp.float32) for _ in range(2 * _NBUF)],
            pltpu.VMEM((_ZR, 64), jnp.float32),
            pltpu.VMEM_SHARED((_NPAD, 64), jnp.float32),
            pltpu.SemaphoreType.DMA,
            pltpu.SemaphoreType.DMA,
            pltpu.SemaphoreType.DMA,
            pltpu.SemaphoreType.DMA,
        ],
    )
    def edge_kernel(ul_hbm, ur_hbm, row_hbm, col_hbm, out_hbm,
                    rowv, colv, bufs, zv, acc, gsem0, gsem1, ssem0, ssem1):
        c = lax.axis_index("c")
        s = lax.axis_index("s")
        _fill2d(zv, _ZR, 64, 0.0)
        for j in range(5):
            pltpu.sync_copy(zv, acc.at[pl.ds(s * _RPT + j * _ZR, _ZR)])
        pltpu.sync_copy(row_hbm.at[s], rowv)
        pltpu.sync_copy(col_hbm.at[s], colv)
        plsc.subcore_barrier()

        def run(u_half):
            def wave(i, carry):
                g = i * _NBUF
                gh = [
                    pltpu.async_copy(u_half.at[rowv.at[g + j]], bufs[j], gsem0)
                    for j in range(_NBUF)
                ]
                for h in gh:
                    h.wait()
                sh = [
                    pltpu.async_copy(bufs[j], acc.at[colv.at[g + j]], ssem0,
                                     add=True)
                    for j in range(_NBUF)
                ]
                for h in sh:
                    h.wait()
                return carry

            lax.fori_loop(0, _TB_S // _NBUF, wave, 0)

        @pl.when(c == 0)
        def _():
            run(ul_hbm)

        @pl.when(c == 1)
        def _():
            run(ur_hbm)

        plsc.subcore_barrier()
        for j in range(5):
            r0 = s * _RPT + j * _ZR
            pltpu.sync_copy(acc.at[pl.ds(r0, _ZR)],
                            out_hbm.at[pl.ds(c * _NPAD + r0, _ZR)])

    return edge_kernel


def _make_edge_scatter16():
    mesh = plsc.VectorSubcoreMesh(core_axis_name="c", subcore_axis_name="s")

    @functools.partial(
        pl.kernel,
        out_type=jax.ShapeDtypeStruct((2 * _NPAD, 16), jnp.float32),
        mesh=mesh,
        compiler_params=pltpu.CompilerParams(use_tc_tiling_on_sc=False),
        scratch_types=[
            pltpu.VMEM((_TB_W, _EBW), jnp.int32),
            pltpu.VMEM((_TB_W, _EBW), jnp.int32),
            [pltpu.VMEM((_EBW, 16), jnp.float32) for _ in range(2 * _NBUF)],
            pltpu.VMEM((_ZR, 16), jnp.float32),
            pltpu.VMEM_SHARED((_NPAD, 16), jnp.float32),
            pltpu.SemaphoreType.DMA,
            pltpu.SemaphoreType.DMA,
            pltpu.SemaphoreType.DMA,
            pltpu.SemaphoreType.DMA,
        ],
    )
    def edge_kernel(u_hbm, row_hbm, col_hbm, out_hbm,
                    rowv, colv, bufs, zv, acc, gsem0, gsem1, ssem0, ssem1):
        c = lax.axis_index("c")
        s = lax.axis_index("s")
        w = c * _NS + s
        _fill2d(zv, _ZR, 16, 0.0)
        for j in range(5):
            pltpu.sync_copy(zv, acc.at[pl.ds(s * _RPT + j * _ZR, _ZR)])
        pltpu.sync_copy(row_hbm.at[w], rowv)
        pltpu.sync_copy(col_hbm.at[w], colv)
        plsc.subcore_barrier()

        sets = (bufs[:_NBUF], bufs[_NBUF:])
        n_groups = (_TB_W // _NBUF - 1) // 2

        def issue_gathers(bset, g, sem):
            for j in range(_NBUF):
                pltpu.async_copy(u_hbm.at[rowv.at[g + j]], bset[j], sem)

        def wait_gathers(bset, g, sem):
            for j in range(_NBUF):
                pltpu.make_async_copy(u_hbm.at[rowv.at[g + j]], bset[j],
                                      sem).wait()

        def do_scatters(bset, g, sem):
            sh = [
                pltpu.async_copy(bset[j], acc.at[colv.at[g + j]], sem, add=True)
                for j in range(_NBUF)
            ]
            for h in sh:
                h.wait()

        issue_gathers(sets[0], 0, gsem0)

        def group(k, carry):
            g0 = 2 * _NBUF * k
            g1 = g0 + _NBUF
            issue_gathers(sets[1], g1, gsem1)
            wait_gathers(sets[0], g0, gsem0)
            do_scatters(sets[0], g0, ssem0)

            issue_gathers(sets[0], g0 + 2 * _NBUF, gsem0)
            wait_gathers(sets[1], g1, gsem1)
            do_scatters(sets[1], g1, ssem1)
            return carry

        lax.fori_loop(0, n_groups, group, 0)
        gt = 2 * _NBUF * n_groups
        wait_gathers(sets[0], gt, gsem0)
        do_scatters(sets[0], gt, ssem0)
        plsc.subcore_barrier()
        for j in range(5):
            r0 = s * _RPT + j * _ZR
            pltpu.sync_copy(acc.at[pl.ds(r0, _ZR)],
                            out_hbm.at[pl.ds(c * _NPAD + r0, _ZR)])

    return edge_kernel


_make_degree_kernel = functools.cache(_make_degree_kernel)
_make_edge_scatter_split = functools.cache(_make_edge_scatter_split)
_make_edge_scatter16 = functools.cache(_make_edge_scatter16)


def _dis_from_deg(d0_blk, d1_blk):
    deg = d0_blk[:, 0:1] + d1_blk[:, 0:1] + 1.0
    return lax.rsqrt(deg)


def _kan_math(h, lng, lnb, wg, wbt, bb):
    m = jnp.mean(h, axis=-1, keepdims=True)
    v = jnp.mean((h - m) ** 2, axis=-1, keepdims=True)
    y = (h - m) * lax.rsqrt(v + _EPS) * lng + lnb
    acc = jnp.dot(h * jax.nn.sigmoid(h), wbt,
                  preferred_element_type=jnp.float32) + bb
    for g in range(_GRIDS):
        bg = jnp.exp(-(((y - _GRIDPTS[g]) * (1.0 / _H)) ** 2))
        acc = acc + jnp.dot(bg, wg[g], preferred_element_type=jnp.float32)
    return acc


def _full_spec(shape):
    n = len(shape)
    return pl.BlockSpec(shape, lambda i, _n=n: (0,) * _n)


def _row_spec(width):
    return pl.BlockSpec((_BR, width), lambda i: (i, 0))


def _tc_kan_first(x, d0, d1, lng, lnb, wg, wbt, bb):
    def body(x_ref, d0_ref, d1_ref, lng_ref, lnb_ref, wg_ref, wbt_ref, bb_ref,
             ul_ref, ur_ref):
        dis = _dis_from_deg(d0_ref[...], d1_ref[...])
        t = _kan_math(x_ref[...], lng_ref[...], lnb_ref[...], wg_ref[...],
                      wbt_ref[...], bb_ref[...])
        u = t * dis
        ul_ref[...] = u[:, :64]
        ur_ref[...] = u[:, 64:]

    return pl.pallas_call(
        body,
        grid=(_GR,),
        in_specs=[
            _row_spec(_D_IN), _row_spec(16), _row_spec(16),
            _full_spec(lng.shape), _full_spec(lnb.shape),
            _full_spec(wg.shape), _full_spec(wbt.shape), _full_spec(bb.shape),
        ],
        out_specs=[_row_spec(64), _row_spec(64)],
        out_shape=[jax.ShapeDtypeStruct((_N, 64), jnp.float32),
                   jax.ShapeDtypeStruct((_N, 64), jnp.float32)],
    )(x, d0, d1, lng, lnb, wg, wbt, bb)


def _tc_combine_stats(ul, ur, sp, d0, d1, bias):
    def body(ul_ref, ur_ref, sl_ref, sr_ref, d0_ref, d1_ref, b_ref,
             h_ref, st_ref):
        i = pl.program_id(0)
        dis = _dis_from_deg(d0_ref[...], d1_ref[...])
        su = jnp.concatenate(
            [sl_ref[...] + ul_ref[...], sr_ref[...] + ur_ref[...]], axis=1)
        h = su * dis + b_ref[...]
        h_ref[...] = h
        new = jnp.concatenate(
            [jnp.sum(h, axis=0, keepdims=True),
             jnp.sum(h * h, axis=0, keepdims=True)], axis=0)

        @pl.when(i == 0)
        def _():
            st_ref[...] = new

        @pl.when(i != 0)
        def _():
            st_ref[...] = st_ref[...] + new

    return pl.pallas_call(
        body,
        grid=(_GR,),
        in_specs=[
            _row_spec(64), _row_spec(64), _row_spec(64), _row_spec(64),
            _row_spec(16), _row_spec(16), _full_spec(bias.shape),
        ],
        out_specs=[_row_spec(_HID), _full_spec((2, _HID))],
        out_shape=[jax.ShapeDtypeStruct((_N, _HID), jnp.float32),
                   jax.ShapeDtypeStruct((2, _HID), jnp.float32)],
    )(ul, ur, sp[:_N], sp[_NPAD:_NPAD + _N], d0, d1, bias)


def _tc_bn_kan(h_pre, st, d0, d1, lng, lnb, wg, wbt, bb, bng, bnb):
    def body(hp_ref, st_ref, d0_ref, d1_ref, lng_ref, lnb_ref, wg_ref,
             wbt_ref, bb_ref, bng_ref, bnb_ref, ht_ref, ul_ref, ur_ref):
        st_v = st_ref[...]
        m = st_v[0:1, :] * (1.0 / _N)
        var = st_v[1:2, :] * (1.0 / _N) - m * m
        ht = (hp_ref[...] - m) * (bng_ref[...] * lax.rsqrt(var + _EPS)) + bnb_ref[...]
        ht_ref[...] = ht
        dis = _dis_from_deg(d0_ref[...], d1_ref[...])
        t = _kan_math(ht, lng_ref[...], lnb_ref[...], wg_ref[...], wbt_ref[...],
                      bb_ref[...])
        u = t * dis
        ul_ref[...] = u[:, :64]
        ur_ref[...] = u[:, 64:]

    return pl.pallas_call(
        body,
        grid=(_GR,),
        in_specs=[
            _row_spec(_HID), _full_spec((2, _HID)), _row_spec(16), _row_spec(16),
            _full_spec(lng.shape), _full_spec(lnb.shape),
            _full_spec(wg.shape), _full_spec(wbt.shape), _full_spec(bb.shape),
            _full_spec(bng.shape), _full_spec(bnb.shape),
        ],
        out_specs=[_row_spec(_HID), _row_spec(64), _row_spec(64)],
        out_shape=[jax.ShapeDtypeStruct((_N, _HID), jnp.float32),
                   jax.ShapeDtypeStruct((_N, 64), jnp.float32),
                   jax.ShapeDtypeStruct((_N, 64), jnp.float32)],
    )(h_pre, st, d0, d1, lng, lnb, wg, wbt, bb, bng, bnb)


def _tc_kan_concat(x, h1, h2_pre, st2, d0, d1, lng, lnb, wg, wbt, bb, bng, bnb):
    def body(x_ref, h1_ref, hp_ref, st_ref, d0_ref, d1_ref, lng_ref, lnb_ref,
             wg_ref, wbt_ref, bb_ref, bng_ref, bnb_ref, u_ref):
        st_v = st_ref[...]
        m = st_v[0:1, :] * (1.0 / _N)
        var = st_v[1:2, :] * (1.0 / _N) - m * m
        ht2 = (hp_ref[...] - m) * (bng_ref[...] * lax.rsqrt(var + _EPS)) + bnb_ref[...]
        h = jnp.concatenate([x_ref[...], h1_ref[...], ht2], axis=1)
        dis = _dis_from_deg(d0_ref[...], d1_ref[...])
        t = _kan_math(h, lng_ref[...], lnb_ref[...], wg_ref[...], wbt_ref[...],
                      bb_ref[...])
        u_ref[...] = t * dis

    return pl.pallas_call(
        body,
        grid=(_GR,),
        in_specs=[
            _row_spec(_D_IN), _row_spec(_HID), _row_spec(_HID),
            _full_spec((2, _HID)), _row_spec(16), _row_spec(16),
            _full_spec(lng.shape), _full_spec(lnb.shape),
            _full_spec(wg.shape), _full_spec(wbt.shape), _full_spec(bb.shape),
            _full_spec(bng.shape), _full_spec(bnb.shape),
        ],
        out_specs=_row_spec(_NCLS),
        out_shape=jax.ShapeDtypeStruct((_N, _NCLS), jnp.float32),
    )(x, h1, h2_pre, st2, d0, d1, lng, lnb, wg, wbt, bb, bng, bnb)


def _tc_combine_final(u, sp, d0, d1, bias):
    def body(u_ref, s0_ref, s1_ref, d0_ref, d1_ref, b_ref, o_ref):
        dis = _dis_from_deg(d0_ref[...], d1_ref[...])
        o_ref[...] = (s0_ref[...] + s1_ref[...] + u_ref[...]) * dis + b_ref[...]

    return pl.pallas_call(
        body,
        grid=(_GR,),
        in_specs=[
            _row_spec(_NCLS), _row_spec(_NCLS), _row_spec(_NCLS),
            _row_spec(16), _row_spec(16), _full_spec(bias.shape),
        ],
        out_specs=_row_spec(_NCLS),
        out_shape=jax.ShapeDtypeStruct((_N, _NCLS), jnp.float32),
    )(u, sp[:_N], sp[_NPAD:_NPAD + _N], d0, d1, bias)


def _prep_kan_weights(Ws, Wb, din, dout):
    wg = Ws.reshape(dout, din, _GRIDS).transpose(2, 1, 0)
    return wg, Wb.T


def kernel(x, edge_index, ln_g1, ln_b1, Ws1, Wb1, bb1, bias1,
           ln_g2, ln_b2, Ws2, Wb2, bb2, bias2,
           ln_g3, ln_b3, Ws3, Wb3, bb3, bias3, bn_g, bn_b):
    row_w = edge_index[0].reshape(_NW, _TB_W, _EBW)
    col_w = edge_index[1].reshape(_NW, _TB_W, _EBW)
    row_s = edge_index[0].reshape(_NS, _TB_S, _EBS)
    col_s = edge_index[1].reshape(_NS, _TB_S, _EBS)

    wg1, wbt1 = _prep_kan_weights(Ws1, Wb1, _D_IN, _HID)
    wg2, wbt2 = _prep_kan_weights(Ws2, Wb2, _HID, _HID)
    wg3, wbt3 = _prep_kan_weights(Ws3, Wb3, _D_IN + 2 * _HID, _NCLS)
    lng1, lnb1 = ln_g1.reshape(1, -1), ln_b1.reshape(1, -1)
    lng2, lnb2 = ln_g2.reshape(1, -1), ln_b2.reshape(1, -1)
    lng3, lnb3 = ln_g3.reshape(1, -1), ln_b3.reshape(1, -1)
    bbr1, bbr2, bbr3 = bb1.reshape(1, -1), bb2.reshape(1, -1), bb3.reshape(1, -1)
    br1, br2, br3 = bias1.reshape(1, -1), bias2.reshape(1, -1), bias3.reshape(1, -1)
    bng, bnb = bn_g.reshape(1, -1), bn_b.reshape(1, -1)

    degp = _make_degree_kernel()(col_w)
    d0, d1 = degp[:_N], degp[_NPAD:_NPAD + _N]

    ul1, ur1 = _tc_kan_first(x, d0, d1, lng1, lnb1, wg1, wbt1, bbr1)
    sp1 = _make_edge_scatter_split()(ul1, ur1, row_s, col_s)
    h1_pre, st1 = _tc_combine_stats(ul1, ur1, sp1, d0, d1, br1)

    h1, ul2, ur2 = _tc_bn_kan(h1_pre, st1, d0, d1, lng2, lnb2, wg2, wbt2,
                              bbr2, bng, bnb)
    sp2 = _make_edge_scatter_split()(ul2, ur2, row_s, col_s)
    h2_pre, st2 = _tc_combine_stats(ul2, ur2, sp2, d0, d1, br2)

    u3 = _tc_kan_concat(x, h1, h2_pre, st2, d0, d1, lng3, lnb3, wg3, wbt3,
                        bbr3, bng, bnb)
    sp3 = _make_edge_scatter16()(u3, row_w, col_w)
    return _tc_combine_final(u3, sp3, d0, d1, br3)
